# baseline (speedup 1.0000x reference)
import functools

import jax
import jax.numpy as jnp
from jax import lax
from jax.experimental import pallas as pl
from jax.experimental.pallas import tpu as pltpu
from jax.experimental.pallas import tpu_sc as plsc

NC = 2
NS = 16
NW = NC * NS
LANE = 16

NPAD = 102400
PER_TILE = NPAD // NS
PER_W = NPAD // NW
ZB = 800
PAD_IDX = NPAD - 8

CHUNK = 1536
ROWS = CHUNK // 128

_C = (4.2548806287011235e-08, 1.4426876043335413, -0.7211320506025966,
      0.47846332485045534, -0.3465458417713943, 0.24040568619683228,
      -0.1359227157447176, 0.05113245146722506, -0.009088543825902295)
_LN2 = 0.6931471805599453


def _log2_poly(m):
    bits = lax.bitcast_convert_type(m, jnp.int32)
    e = (bits >> 23) - 127
    f = lax.bitcast_convert_type((bits & 0x007FFFFF) | 0x3F800000,
                                 jnp.float32)
    r = f - 1.0
    p = jnp.full_like(r, _C[8])
    for c in _C[7::-1]:
        p = p * r + c
    return e.astype(jnp.float32) + p


def _ploop(n, fn):
    plsc.parallel_loop(0, n, unroll=4)(fn)


def _axis_ids():
    return lax.axis_index("c"), lax.axis_index("s")


def _dma_wait(d):
    d.wait()


def _scatter_add(acc_sh, idx_row_ref, val_row_ref, sem):
    return pltpu.async_copy(val_row_ref, acc_sh.at[idx_row_ref], sem, add=True)


def _drain_scatter(acc_sh, idx_row_ref, val_row_ref, sem):
    pltpu.make_async_copy(val_row_ref, acc_sh.at[idx_row_ref], sem).wait()


def _edge_body(E, edge_ref, probs_ref, x_ref, w_ref, part_ref,
               acc_sh, x_v, w_v, eb0, eb1, eb2, pb0, pb1, pb2,
               vb0, vb1, vb2, zb, sl0, sl1, sl2, ss0, ss1, ss2):
    EB = (eb0, eb1, eb2)
    PB = (pb0, pb1, pb2)
    VB = (vb0, vb1, vb2)
    SL = (sl0, sl1, sl2)
    SS = (ss0, ss1, ss2)

    EW = (E // NW) // 128 * 128
    rem = E - NW * EW
    n_full = EW // CHUNK
    tail = EW % CHUNK
    n_chunks = n_full + (1 if tail else 0)
    t_rows = tail // 128
    t_rem = tail % 128
    t_nrows = t_rows + (1 if t_rem else 0)

    c, s = _axis_ids()
    wid = c * NS + s
    tec_base = wid * EW

    xd = pltpu.async_copy(x_ref, x_v, SL[0])
    pltpu.sync_copy(w_ref, w_v)

    def zfill(i):
        zb[pl.ds(i * LANE, LANE)] = jnp.zeros((LANE,), jnp.float32)
    _ploop(ZB // LANE, zfill)
    for r in range(PER_TILE // ZB):
        pltpu.sync_copy(zb, acc_sh.at[pl.ds(s * PER_TILE + r * ZB, ZB)])
    _dma_wait(xd)
    plsc.subcore_barrier()

    w_vec = w_v[...]

    def in_descs_abs(base, b, n_words):
        return (pltpu.make_async_copy(edge_ref.at[:, pl.ds(base, n_words)],
                                      EB[b].at[:, pl.ds(0, n_words)], SL[b]),
                pltpu.make_async_copy(probs_ref.at[pl.ds(base, n_words)],
                                      PB[b].at[pl.ds(0, n_words)], SL[b]))

    def in_descs(ci, b, n_words):
        return in_descs_abs(tec_base + ci * CHUNK, b, n_words)

    def compute(b, n_groups):
        eb, pb, vb = EB[b], PB[b], VB[b]

        def grp(g):
            idx = eb[0, pl.ds(g * LANE, LANE)]
            xs = plsc.load_gather(x_v, [idx])
            pv = pb[pl.ds(g * LANE, LANE)]
            m = 1.0 - xs * pv * w_vec
            vb[g >> 3, pl.ds((g & 7) * LANE, LANE)] = _log2_poly(m)
        _ploop(n_groups, grp)

    def fire(b, nrows):
        return [_scatter_add(acc_sh, EB[b].at[1, pl.ds(j * 128, 128)],
                             VB[b].at[j], SS[b]) for j in range(nrows)]

    def drain(b, nrows):
        for j in range(nrows):
            _drain_scatter(acc_sh, EB[b].at[1, pl.ds(j * 128, 128)],
                           VB[b].at[j], SS[b])

    def pwords(ci):
        return tail if (tail and ci == n_chunks - 1) else CHUNK

    def phase(ci, b, prefetch_ci, do_drain, prefetch_words=CHUNK):
        for d in in_descs(ci, b, CHUNK):
            _dma_wait(d)
        nb = (b + 1) % 3
        if do_drain:
            drain(nb, ROWS)
        if prefetch_ci is not None:
            for d in in_descs(prefetch_ci, nb, prefetch_words):
                d.start()
        compute(b, CHUNK // LANE)
        fire(b, ROWS)

    if rem:
        for start in range(0, rem, CHUNK):
            n = min(CHUNK, rem - start)
            nr, nrem = n // 128, n % 128

            @pl.when(wid == 0)
            def _():
                ds_ = in_descs_abs(NW * EW + start, 2, n)
                for d in ds_:
                    d.start()
                for d in ds_:
                    _dma_wait(d)
                if nrem:
                    for k in range(nrem, 128, LANE):
                        EB[2][1, pl.ds(nr * 128 + k, LANE)] = jnp.full(
                            (LANE,), PAD_IDX, jnp.int32)
                compute(2, n // LANE)
                fired = fire(2, nr + (1 if nrem else 0))
                for d in fired:
                    _dma_wait(d)

    n_trip = max(0, (n_chunks - 4) // 3)

    for d in in_descs(0, 0, pwords(0)):
        d.start()
    if n_chunks >= 2:
        phase(0, 0, 1, False, pwords(1))
    if n_chunks >= 3:
        phase(1, 1, 2, False, pwords(2))

    def triple(i, _):
        ci = 2 + i * 3
        phase(ci, 2, ci + 1, True)
        phase(ci + 1, 0, ci + 2, True)
        phase(ci + 2, 1, ci + 3, True)
        return 0
    lax.fori_loop(0, n_trip, triple, 0)

    for ci in range(2 + 3 * n_trip, n_chunks - 1):
        phase(ci, ci % 3, ci + 1, True, pwords(ci + 1))

    ci = n_chunks - 1
    b = ci % 3
    lw = tail if tail else CHUNK
    for d in in_descs(ci, b, lw):
        _dma_wait(d)
    if n_chunks >= 3:
        drain((b + 1) % 3, ROWS)
    if t_rem:
        for k in range(t_rem, 128, LANE):
            EB[b][1, pl.ds(t_rows * 128 + k, LANE)] = jnp.full(
                (LANE,), PAD_IDX, jnp.int32)
    compute(b, lw // LANE)
    last = fire(b, t_nrows if tail else ROWS)

    if n_chunks >= 2:
        drain((b + 2) % 3, ROWS)
    for d in last:
        _dma_wait(d)

    plsc.subcore_barrier()
    pltpu.sync_copy(acc_sh.at[pl.ds(s * PER_TILE, PER_TILE)],
                    part_ref.at[pl.ds(c * NPAD + s * PER_TILE, PER_TILE)])


def _combine_body(n_nodes, part_ref, x_ref, sl_ref, out_ref,
                  p_v, x_v, o_v, sl_v):
    c, s = _axis_ids()
    wid = c * NS + s
    base = wid * PER_W
    last_n = n_nodes - (NW - 1) * PER_W
    for cc in range(NC):
        pltpu.sync_copy(part_ref.at[pl.ds(cc * NPAD + base, PER_W)],
                        p_v.at[cc])
    if last_n == PER_W:
        pltpu.sync_copy(x_ref.at[pl.ds(base, PER_W)], x_v)
    else:
        @pl.when(wid < NW - 1)
        def _():
            pltpu.sync_copy(x_ref.at[pl.ds(base, PER_W)], x_v)

        @pl.when(wid == NW - 1)
        def _():
            pltpu.sync_copy(x_ref.at[pl.ds(base, last_n)],
                            x_v.at[pl.ds(0, last_n)])
    pltpu.sync_copy(sl_ref, sl_v)
    sl_vec = sl_v[...]

    def grp(g):
        d = pl.ds(g * LANE, LANE)
        lg = p_v[0, d]
        for cc in range(1, NC):
            lg = lg + p_v[cc, d]
        prod = jnp.exp(lg * _LN2)
        o_v[d] = prod + sl_vec * x_v[d]
    _ploop(PER_W // LANE, grp)
    if last_n == PER_W:
        pltpu.sync_copy(o_v, out_ref.at[pl.ds(base, PER_W)])
    else:
        @pl.when(wid < NW - 1)
        def _():
            pltpu.sync_copy(o_v, out_ref.at[pl.ds(base, PER_W)])

        @pl.when(wid == NW - 1)
        def _():
            pltpu.sync_copy(o_v.at[pl.ds(0, last_n)],
                            out_ref.at[pl.ds(base, last_n)])


def kernel(x, edge_index, edge_probs, edge_weight, self_loop):
    n_nodes = x.shape[0]
    E = edge_index.shape[1]
    f32 = jnp.float32
    x_flat = x.reshape(n_nodes).astype(f32)
    w16 = jnp.broadcast_to(edge_weight.astype(f32).reshape(1), (LANE,))
    sl16 = jnp.broadcast_to(jnp.asarray(self_loop, f32).reshape(1), (LANE,))
    mesh = plsc.VectorSubcoreMesh(core_axis_name="c", subcore_axis_name="s",
                                  num_cores=NC, num_subcores=NS)

    edge_k = pl.kernel(
        functools.partial(_edge_body, E),
        out_type=jax.ShapeDtypeStruct((NC * NPAD,), f32),
        mesh=mesh,
        compiler_params=pltpu.CompilerParams(needs_layout_passes=False),
        scratch_types=[
            pltpu.VMEM_SHARED((NPAD,), f32),
            pltpu.VMEM((n_nodes,), f32),
            pltpu.VMEM((LANE,), f32),
            pltpu.VMEM((2, CHUNK), jnp.int32),
            pltpu.VMEM((2, CHUNK), jnp.int32),
            pltpu.VMEM((2, CHUNK), jnp.int32),
            pltpu.VMEM((CHUNK,), f32),
            pltpu.VMEM((CHUNK,), f32),
            pltpu.VMEM((CHUNK,), f32),
            pltpu.VMEM((ROWS, 128), f32),
            pltpu.VMEM((ROWS, 128), f32),
            pltpu.VMEM((ROWS, 128), f32),
            pltpu.VMEM((ZB,), f32),
            pltpu.SemaphoreType.DMA,
            pltpu.SemaphoreType.DMA,
            pltpu.SemaphoreType.DMA,
            pltpu.SemaphoreType.DMA,
            pltpu.SemaphoreType.DMA,
            pltpu.SemaphoreType.DMA,
        ],
    )
    partial = edge_k(edge_index, edge_probs, x_flat, w16)

    combine_k = pl.kernel(
        functools.partial(_combine_body, n_nodes),
        out_type=jax.ShapeDtypeStruct((n_nodes,), f32),
        mesh=mesh,
        compiler_params=pltpu.CompilerParams(needs_layout_passes=False),
        scratch_types=[
            pltpu.VMEM((NC, PER_W), f32),
            pltpu.VMEM((PER_W,), f32),
            pltpu.VMEM((PER_W,), f32),
            pltpu.VMEM((LANE,), f32),
        ],
    )
    out_flat = combine_k(partial, x_flat, sl16)
    return out_flat.reshape(n_nodes, 1).astype(x.dtype)


# speedup vs baseline: 1.0748x; 1.0748x over previous
import functools

import jax
import jax.numpy as jnp
from jax import lax
from jax.experimental import pallas as pl
from jax.experimental.pallas import tpu as pltpu
from jax.experimental.pallas import tpu_sc as plsc

NC = 2
NS = 16
NW = NC * NS
LANE = 16

NPAD = 102400
PER_TILE = NPAD // NS
PER_W = NPAD // NW
ZB = 800
PAD_IDX = NPAD - 8

CHUNK = 1792
ROWS = CHUNK // 128

_C = (4.2548806287011235e-08, 1.4426876043335413, -0.7211320506025966,
      0.47846332485045534, -0.3465458417713943, 0.24040568619683228,
      -0.1359227157447176, 0.05113245146722506, -0.009088543825902295)
_LN2 = 0.6931471805599453


def _log2_poly(m):
    bits = lax.bitcast_convert_type(m, jnp.int32)
    e = (bits >> 23) - 127
    f = lax.bitcast_convert_type((bits & 0x007FFFFF) | 0x3F800000,
                                 jnp.float32)
    r = f - 1.0
    p = jnp.full_like(r, _C[8])
    for c in _C[7::-1]:
        p = p * r + c
    return e.astype(jnp.float32) + p


def _ploop(n, fn):
    plsc.parallel_loop(0, n, unroll=4)(fn)


def _axis_ids():
    return lax.axis_index("c"), lax.axis_index("s")


def _dma_wait(d):
    d.wait()


def _scatter_add(acc_sh, idx_row_ref, val_row_ref, sem):
    return pltpu.async_copy(val_row_ref, acc_sh.at[idx_row_ref], sem, add=True)


def _drain_scatter(acc_sh, idx_row_ref, val_row_ref, sem):
    pltpu.make_async_copy(val_row_ref, acc_sh.at[idx_row_ref], sem).wait()


def _edge_body(E, edge_ref, probs_ref, x_ref, w_ref, part_ref,
               acc_sh, x_v, w_v, eb0, eb1, eb2, pb0, pb1, pb2,
               vb0, vb1, vb2, zb, sl0, sl1, sl2, ss0, ss1, ss2):
    EB = (eb0, eb1, eb2)
    PB = (pb0, pb1, pb2)
    VB = (vb0, vb1, vb2)
    SL = (sl0, sl1, sl2)
    SS = (ss0, ss1, ss2)

    EW = (E // NW) // 128 * 128
    rem = E - NW * EW
    n_full = EW // CHUNK
    tail = EW % CHUNK
    n_chunks = n_full + (1 if tail else 0)
    t_rows = tail // 128
    t_rem = tail % 128
    t_nrows = t_rows + (1 if t_rem else 0)

    c, s = _axis_ids()
    wid = c * NS + s
    tec_base = wid * EW

    xd = pltpu.async_copy(x_ref, x_v, SL[0])
    pltpu.sync_copy(w_ref, w_v)

    def zfill(i):
        zb[pl.ds(i * LANE, LANE)] = jnp.zeros((LANE,), jnp.float32)
    _ploop(ZB // LANE, zfill)
    for r in range(PER_TILE // ZB):
        pltpu.sync_copy(zb, acc_sh.at[pl.ds(s * PER_TILE + r * ZB, ZB)])
    _dma_wait(xd)
    plsc.subcore_barrier()

    w_vec = w_v[...]

    def in_descs_abs(base, b, n_words):
        return (pltpu.make_async_copy(edge_ref.at[:, pl.ds(base, n_words)],
                                      EB[b].at[:, pl.ds(0, n_words)], SL[b]),
                pltpu.make_async_copy(probs_ref.at[pl.ds(base, n_words)],
                                      PB[b].at[pl.ds(0, n_words)], SL[b]))

    def in_descs(ci, b, n_words):
        return in_descs_abs(tec_base + ci * CHUNK, b, n_words)

    def compute(b, n_groups):
        eb, pb, vb = EB[b], PB[b], VB[b]

        def grp(g):
            idx = eb[0, pl.ds(g * LANE, LANE)]
            xs = plsc.load_gather(x_v, [idx])
            pv = pb[pl.ds(g * LANE, LANE)]
            m = 1.0 - xs * pv * w_vec
            vb[g >> 3, pl.ds((g & 7) * LANE, LANE)] = _log2_poly(m)
        _ploop(n_groups, grp)

    def fire(b, nrows):
        return [_scatter_add(acc_sh, EB[b].at[1, pl.ds(j * 128, 128)],
                             VB[b].at[j], SS[b]) for j in range(nrows)]

    def drain(b, nrows):
        for j in range(nrows):
            _drain_scatter(acc_sh, EB[b].at[1, pl.ds(j * 128, 128)],
                           VB[b].at[j], SS[b])

    def pwords(ci):
        return tail if (tail and ci == n_chunks - 1) else CHUNK

    def phase(ci, b, prefetch_ci, do_drain, prefetch_words=CHUNK):
        for d in in_descs(ci, b, CHUNK):
            _dma_wait(d)
        nb = (b + 1) % 3
        if do_drain:
            drain(nb, ROWS)
        if prefetch_ci is not None:
            for d in in_descs(prefetch_ci, nb, prefetch_words):
                d.start()
        compute(b, CHUNK // LANE)
        fire(b, ROWS)

    if rem:
        for start in range(0, rem, CHUNK):
            n = min(CHUNK, rem - start)
            nr, nrem = n // 128, n % 128

            @pl.when(wid == 0)
            def _():
                ds_ = in_descs_abs(NW * EW + start, 2, n)
                for d in ds_:
                    d.start()
                for d in ds_:
                    _dma_wait(d)
                if nrem:
                    for k in range(nrem, 128, LANE):
                        EB[2][1, pl.ds(nr * 128 + k, LANE)] = jnp.full(
                            (LANE,), PAD_IDX, jnp.int32)
                compute(2, n // LANE)
                fired = fire(2, nr + (1 if nrem else 0))
                for d in fired:
                    _dma_wait(d)

    n_trip = max(0, (n_chunks - 4) // 3)

    for d in in_descs(0, 0, pwords(0)):
        d.start()
    if n_chunks >= 2:
        phase(0, 0, 1, False, pwords(1))
    if n_chunks >= 3:
        phase(1, 1, 2, False, pwords(2))

    def triple(i, _):
        ci = 2 + i * 3
        phase(ci, 2, ci + 1, True)
        phase(ci + 1, 0, ci + 2, True)
        phase(ci + 2, 1, ci + 3, True)
        return 0
    lax.fori_loop(0, n_trip, triple, 0)

    for ci in range(2 + 3 * n_trip, n_chunks - 1):
        phase(ci, ci % 3, ci + 1, True, pwords(ci + 1))

    ci = n_chunks - 1
    b = ci % 3
    lw = tail if tail else CHUNK
    for d in in_descs(ci, b, lw):
        _dma_wait(d)
    if n_chunks >= 3:
        drain((b + 1) % 3, ROWS)
    if t_rem:
        for k in range(t_rem, 128, LANE):
            EB[b][1, pl.ds(t_rows * 128 + k, LANE)] = jnp.full(
                (LANE,), PAD_IDX, jnp.int32)
    compute(b, lw // LANE)
    last = fire(b, t_nrows if tail else ROWS)

    if n_chunks >= 2:
        drain((b + 2) % 3, ROWS)
    for d in last:
        _dma_wait(d)

    plsc.subcore_barrier()
    pltpu.sync_copy(acc_sh.at[pl.ds(s * PER_TILE, PER_TILE)],
                    part_ref.at[pl.ds(c * NPAD + s * PER_TILE, PER_TILE)])


def _combine_body(n_nodes, part_ref, x_ref, sl_ref, out_ref,
                  p_v, x_v, o_v, sl_v):
    c, s = _axis_ids()
    wid = c * NS + s
    base = wid * PER_W
    last_n = n_nodes - (NW - 1) * PER_W
    for cc in range(NC):
        pltpu.sync_copy(part_ref.at[pl.ds(cc * NPAD + base, PER_W)],
                        p_v.at[cc])
    if last_n == PER_W:
        pltpu.sync_copy(x_ref.at[pl.ds(base, PER_W)], x_v)
    else:
        @pl.when(wid < NW - 1)
        def _():
            pltpu.sync_copy(x_ref.at[pl.ds(base, PER_W)], x_v)

        @pl.when(wid == NW - 1)
        def _():
            pltpu.sync_copy(x_ref.at[pl.ds(base, last_n)],
                            x_v.at[pl.ds(0, last_n)])
    pltpu.sync_copy(sl_ref, sl_v)
    sl_vec = sl_v[...]

    def grp(g):
        d = pl.ds(g * LANE, LANE)
        lg = p_v[0, d]
        for cc in range(1, NC):
            lg = lg + p_v[cc, d]
        prod = jnp.exp(lg * _LN2)
        o_v[d] = prod + sl_vec * x_v[d]
    _ploop(PER_W // LANE, grp)
    if last_n == PER_W:
        pltpu.sync_copy(o_v, out_ref.at[pl.ds(base, PER_W)])
    else:
        @pl.when(wid < NW - 1)
        def _():
            pltpu.sync_copy(o_v, out_ref.at[pl.ds(base, PER_W)])

        @pl.when(wid == NW - 1)
        def _():
            pltpu.sync_copy(o_v.at[pl.ds(0, last_n)],
                            out_ref.at[pl.ds(base, last_n)])


def kernel(x, edge_index, edge_probs, edge_weight, self_loop):
    n_nodes = x.shape[0]
    E = edge_index.shape[1]
    f32 = jnp.float32
    x_flat = x.reshape(n_nodes).astype(f32)
    w16 = jnp.broadcast_to(edge_weight.astype(f32).reshape(1), (LANE,))
    sl16 = jnp.broadcast_to(jnp.asarray(self_loop, f32).reshape(1), (LANE,))
    mesh = plsc.VectorSubcoreMesh(core_axis_name="c", subcore_axis_name="s",
                                  num_cores=NC, num_subcores=NS)

    edge_k = pl.kernel(
        functools.partial(_edge_body, E),
        out_type=jax.ShapeDtypeStruct((NC * NPAD,), f32),
        mesh=mesh,
        compiler_params=pltpu.CompilerParams(needs_layout_passes=False),
        scratch_types=[
            pltpu.VMEM_SHARED((NPAD,), f32),
            pltpu.VMEM((n_nodes,), f32),
            pltpu.VMEM((LANE,), f32),
            pltpu.VMEM((2, CHUNK), jnp.int32),
            pltpu.VMEM((2, CHUNK), jnp.int32),
            pltpu.VMEM((2, CHUNK), jnp.int32),
            pltpu.VMEM((CHUNK,), f32),
            pltpu.VMEM((CHUNK,), f32),
            pltpu.VMEM((CHUNK,), f32),
            pltpu.VMEM((ROWS, 128), f32),
            pltpu.VMEM((ROWS, 128), f32),
            pltpu.VMEM((ROWS, 128), f32),
            pltpu.VMEM((ZB,), f32),
            pltpu.SemaphoreType.DMA,
            pltpu.SemaphoreType.DMA,
            pltpu.SemaphoreType.DMA,
            pltpu.SemaphoreType.DMA,
            pltpu.SemaphoreType.DMA,
            pltpu.SemaphoreType.DMA,
        ],
    )
    partial = edge_k(edge_index, edge_probs, x_flat, w16)

    combine_k = pl.kernel(
        functools.partial(_combine_body, n_nodes),
        out_type=jax.ShapeDtypeStruct((n_nodes,), f32),
        mesh=mesh,
        compiler_params=pltpu.CompilerParams(needs_layout_passes=False),
        scratch_types=[
            pltpu.VMEM((NC, PER_W), f32),
            pltpu.VMEM((PER_W,), f32),
            pltpu.VMEM((PER_W,), f32),
            pltpu.VMEM((LANE,), f32),
        ],
    )
    out_flat = combine_k(partial, x_flat, sl16)
    return out_flat.reshape(n_nodes, 1).astype(x.dtype)


# speedup vs baseline: 1.0967x; 1.0204x over previous
import functools

import jax
import jax.numpy as jnp
from jax import lax
from jax.experimental import pallas as pl
from jax.experimental.pallas import tpu as pltpu
from jax.experimental.pallas import tpu_sc as plsc

NC = 2
NS = 16
NW = NC * NS
LANE = 16

NPAD = 102400
PER_TILE = NPAD // NS
PER_W = NPAD // NW
ZB = 800
PAD_IDX = NPAD - 8

CHUNK = 1920
ROWS = CHUNK // 128

_C = (4.2548806287011235e-08, 1.4426876043335413, -0.7211320506025966,
      0.47846332485045534, -0.3465458417713943, 0.24040568619683228,
      -0.1359227157447176, 0.05113245146722506, -0.009088543825902295)
_LN2 = 0.6931471805599453


def _log2_poly(m):
    bits = lax.bitcast_convert_type(m, jnp.int32)
    e = (bits >> 23) - 127
    f = lax.bitcast_convert_type((bits & 0x007FFFFF) | 0x3F800000,
                                 jnp.float32)
    r = f - 1.0
    p = jnp.full_like(r, _C[8])
    for c in _C[7::-1]:
        p = p * r + c
    return e.astype(jnp.float32) + p


def _ploop(n, fn):
    plsc.parallel_loop(0, n, unroll=4)(fn)


def _axis_ids():
    return lax.axis_index("c"), lax.axis_index("s")


def _dma_wait(d):
    d.wait()


def _scatter_add(acc_sh, idx_row_ref, val_row_ref, sem):
    return pltpu.async_copy(val_row_ref, acc_sh.at[idx_row_ref], sem, add=True)


def _drain_scatter(acc_sh, idx_row_ref, val_row_ref, sem):
    pltpu.make_async_copy(val_row_ref, acc_sh.at[idx_row_ref], sem).wait()


def _edge_body(E, edge_ref, probs_ref, x_ref, w_ref, part_ref,
               acc_sh, x_v, w_v, eb0, eb1, eb2, pb0, pb1, pb2,
               vb0, vb1, vb2, zb, sl0, sl1, sl2, ss0, ss1, ss2):
    EB = (eb0, eb1, eb2)
    PB = (pb0, pb1, pb2)
    VB = (vb0, vb1, vb2)
    SL = (sl0, sl1, sl2)
    SS = (ss0, ss1, ss2)

    EW = (E // NW) // 128 * 128
    rem = E - NW * EW
    n_full = EW // CHUNK
    tail = EW % CHUNK
    n_chunks = n_full + (1 if tail else 0)
    t_rows = tail // 128
    t_rem = tail % 128
    t_nrows = t_rows + (1 if t_rem else 0)

    c, s = _axis_ids()
    wid = c * NS + s
    tec_base = wid * EW

    xd = pltpu.async_copy(x_ref, x_v, SL[0])
    pltpu.sync_copy(w_ref, w_v)

    def zfill(i):
        zb[pl.ds(i * LANE, LANE)] = jnp.zeros((LANE,), jnp.float32)
    _ploop(ZB // LANE, zfill)
    for r in range(PER_TILE // ZB):
        pltpu.sync_copy(zb, acc_sh.at[pl.ds(s * PER_TILE + r * ZB, ZB)])
    _dma_wait(xd)
    plsc.subcore_barrier()

    w_vec = w_v[...]

    def in_descs_abs(base, b, n_words):
        return (pltpu.make_async_copy(edge_ref.at[:, pl.ds(base, n_words)],
                                      EB[b].at[:, pl.ds(0, n_words)], SL[b]),
                pltpu.make_async_copy(probs_ref.at[pl.ds(base, n_words)],
                                      PB[b].at[pl.ds(0, n_words)], SL[b]))

    def in_descs(ci, b, n_words):
        return in_descs_abs(tec_base + ci * CHUNK, b, n_words)

    def compute(b, n_groups):
        eb, pb, vb = EB[b], PB[b], VB[b]

        def grp(g):
            idx = eb[0, pl.ds(g * LANE, LANE)]
            xs = plsc.load_gather(x_v, [idx])
            pv = pb[pl.ds(g * LANE, LANE)]
            m = 1.0 - xs * pv * w_vec
            vb[g >> 3, pl.ds((g & 7) * LANE, LANE)] = _log2_poly(m)
        _ploop(n_groups, grp)

    def fire(b, nrows):
        return [_scatter_add(acc_sh, EB[b].at[1, pl.ds(j * 128, 128)],
                             VB[b].at[j], SS[b]) for j in range(nrows)]

    def drain(b, nrows):
        for j in range(nrows):
            _drain_scatter(acc_sh, EB[b].at[1, pl.ds(j * 128, 128)],
                           VB[b].at[j], SS[b])

    def pwords(ci):
        return tail if (tail and ci == n_chunks - 1) else CHUNK

    def phase(ci, b, prefetch_ci, do_drain, prefetch_words=CHUNK):
        for d in in_descs(ci, b, CHUNK):
            _dma_wait(d)
        nb = (b + 1) % 3
        if do_drain:
            drain(nb, ROWS)
        if prefetch_ci is not None:
            for d in in_descs(prefetch_ci, nb, prefetch_words):
                d.start()
        compute(b, CHUNK // LANE)
        fire(b, ROWS)

    if rem:
        for start in range(0, rem, CHUNK):
            n = min(CHUNK, rem - start)
            nr, nrem = n // 128, n % 128

            @pl.when(wid == 0)
            def _():
                ds_ = in_descs_abs(NW * EW + start, 2, n)
                for d in ds_:
                    d.start()
                for d in ds_:
                    _dma_wait(d)
                if nrem:
                    for k in range(nrem, 128, LANE):
                        EB[2][1, pl.ds(nr * 128 + k, LANE)] = jnp.full(
                            (LANE,), PAD_IDX, jnp.int32)
                compute(2, n // LANE)
                fired = fire(2, nr + (1 if nrem else 0))
                for d in fired:
                    _dma_wait(d)

    n_trip = max(0, (n_chunks - 4) // 3)

    for d in in_descs(0, 0, pwords(0)):
        d.start()
    if n_chunks >= 2:
        phase(0, 0, 1, False, pwords(1))
    if n_chunks >= 3:
        phase(1, 1, 2, False, pwords(2))

    def triple(i, _):
        ci = 2 + i * 3
        phase(ci, 2, ci + 1, True)
        phase(ci + 1, 0, ci + 2, True)
        phase(ci + 2, 1, ci + 3, True)
        return 0
    lax.fori_loop(0, n_trip, triple, 0)

    for ci in range(2 + 3 * n_trip, n_chunks - 1):
        phase(ci, ci % 3, ci + 1, True, pwords(ci + 1))

    ci = n_chunks - 1
    b = ci % 3
    lw = tail if tail else CHUNK
    for d in in_descs(ci, b, lw):
        _dma_wait(d)
    if n_chunks >= 3:
        drain((b + 1) % 3, ROWS)
    if t_rem:
        for k in range(t_rem, 128, LANE):
            EB[b][1, pl.ds(t_rows * 128 + k, LANE)] = jnp.full(
                (LANE,), PAD_IDX, jnp.int32)
    compute(b, lw // LANE)
    last = fire(b, t_nrows if tail else ROWS)

    if n_chunks >= 2:
        drain((b + 2) % 3, ROWS)
    for d in last:
        _dma_wait(d)

    plsc.subcore_barrier()
    pltpu.sync_copy(acc_sh.at[pl.ds(s * PER_TILE, PER_TILE)],
                    part_ref.at[pl.ds(c * NPAD + s * PER_TILE, PER_TILE)])


def _combine_body(n_nodes, part_ref, x_ref, sl_ref, out_ref,
                  p_v, x_v, o_v, sl_v):
    c, s = _axis_ids()
    wid = c * NS + s
    base = wid * PER_W
    last_n = n_nodes - (NW - 1) * PER_W
    for cc in range(NC):
        pltpu.sync_copy(part_ref.at[pl.ds(cc * NPAD + base, PER_W)],
                        p_v.at[cc])
    if last_n == PER_W:
        pltpu.sync_copy(x_ref.at[pl.ds(base, PER_W)], x_v)
    else:
        @pl.when(wid < NW - 1)
        def _():
            pltpu.sync_copy(x_ref.at[pl.ds(base, PER_W)], x_v)

        @pl.when(wid == NW - 1)
        def _():
            pltpu.sync_copy(x_ref.at[pl.ds(base, last_n)],
                            x_v.at[pl.ds(0, last_n)])
    pltpu.sync_copy(sl_ref, sl_v)
    sl_vec = sl_v[...]

    def grp(g):
        d = pl.ds(g * LANE, LANE)
        lg = p_v[0, d]
        for cc in range(1, NC):
            lg = lg + p_v[cc, d]
        prod = jnp.exp(lg * _LN2)
        o_v[d] = prod + sl_vec * x_v[d]
    _ploop(PER_W // LANE, grp)
    if last_n == PER_W:
        pltpu.sync_copy(o_v, out_ref.at[pl.ds(base, PER_W)])
    else:
        @pl.when(wid < NW - 1)
        def _():
            pltpu.sync_copy(o_v, out_ref.at[pl.ds(base, PER_W)])

        @pl.when(wid == NW - 1)
        def _():
            pltpu.sync_copy(o_v.at[pl.ds(0, last_n)],
                            out_ref.at[pl.ds(base, last_n)])


def kernel(x, edge_index, edge_probs, edge_weight, self_loop):
    n_nodes = x.shape[0]
    E = edge_index.shape[1]
    f32 = jnp.float32
    x_flat = x.reshape(n_nodes).astype(f32)
    w16 = jnp.broadcast_to(edge_weight.astype(f32).reshape(1), (LANE,))
    sl16 = jnp.broadcast_to(jnp.asarray(self_loop, f32).reshape(1), (LANE,))
    mesh = plsc.VectorSubcoreMesh(core_axis_name="c", subcore_axis_name="s",
                                  num_cores=NC, num_subcores=NS)

    edge_k = pl.kernel(
        functools.partial(_edge_body, E),
        out_type=jax.ShapeDtypeStruct((NC * NPAD,), f32),
        mesh=mesh,
        compiler_params=pltpu.CompilerParams(needs_layout_passes=False),
        scratch_types=[
            pltpu.VMEM_SHARED((NPAD,), f32),
            pltpu.VMEM((n_nodes,), f32),
            pltpu.VMEM((LANE,), f32),
            pltpu.VMEM((2, CHUNK), jnp.int32),
            pltpu.VMEM((2, CHUNK), jnp.int32),
            pltpu.VMEM((2, CHUNK), jnp.int32),
            pltpu.VMEM((CHUNK,), f32),
            pltpu.VMEM((CHUNK,), f32),
            pltpu.VMEM((CHUNK,), f32),
            pltpu.VMEM((ROWS, 128), f32),
            pltpu.VMEM((ROWS, 128), f32),
            pltpu.VMEM((ROWS, 128), f32),
            pltpu.VMEM((ZB,), f32),
            pltpu.SemaphoreType.DMA,
            pltpu.SemaphoreType.DMA,
            pltpu.SemaphoreType.DMA,
            pltpu.SemaphoreType.DMA,
            pltpu.SemaphoreType.DMA,
            pltpu.SemaphoreType.DMA,
        ],
    )
    partial = edge_k(edge_index, edge_probs, x_flat, w16)

    combine_k = pl.kernel(
        functools.partial(_combine_body, n_nodes),
        out_type=jax.ShapeDtypeStruct((n_nodes,), f32),
        mesh=mesh,
        compiler_params=pltpu.CompilerParams(needs_layout_passes=False),
        scratch_types=[
            pltpu.VMEM((NC, PER_W), f32),
            pltpu.VMEM((PER_W,), f32),
            pltpu.VMEM((PER_W,), f32),
            pltpu.VMEM((LANE,), f32),
        ],
    )
    out_flat = combine_k(partial, x_flat, sl16)
    return out_flat.reshape(n_nodes, 1).astype(x.dtype)
